# baseline (speedup 1.0000x reference)
import functools

import jax
import jax.numpy as jnp
from jax import lax
from jax.experimental import pallas as pl
from jax.experimental.pallas import tpu as pltpu
from jax.experimental.pallas import tpu_sc as plsc

N = 100000
D = 512
S = 15000

L = 16
NC = 2
NS = 16
NW = NC * NS
CHUNK = 3200
NPAD = NW * CHUNK
S_FULL = (S // L) * L

ROWS_BLK = 2048
NBLK = (N + ROWS_BLK - 1) // ROWS_BLK


def _mask_sc_body(seeds_hbm, mask_hbm, seeds_v, mask_v):
    wid = lax.axis_index("s") * NC + lax.axis_index("c")
    base = wid * CHUNK
    pltpu.sync_copy(seeds_hbm, seeds_v)

    ones = jnp.ones((L,), jnp.float32)
    zeros = jnp.zeros((L,), jnp.float32)

    @plsc.parallel_loop(0, CHUNK, L, unroll=8)
    def _init(i):
        mask_v[pl.ds(i, L)] = ones

    def _scatter_group(g):
        s = seeds_v[pl.ds(g, L)]
        local = s - base
        inr = (local >= 0) & (local < CHUNK)
        idx = jnp.where(inr, local, 0)
        plsc.store_scatter(mask_v, [idx], zeros, mask=inr)

    @plsc.parallel_loop(0, S_FULL, L, unroll=8)
    def _scan(g):
        _scatter_group(g)

    _scatter_group(S - L)

    pltpu.sync_copy(mask_v, mask_hbm.at[pl.ds(base, CHUNK)])


def _build_mask(seeds_padded):
    mesh = plsc.VectorSubcoreMesh(core_axis_name="c", subcore_axis_name="s")
    return pl.kernel(
        _mask_sc_body,
        mesh=mesh,
        out_type=jax.ShapeDtypeStruct((NPAD,), jnp.float32),
        scratch_types=[
            pltpu.VMEM((S,), jnp.int32),
            pltpu.VMEM((CHUNK,), jnp.float32),
        ],
        compiler_params=pltpu.CompilerParams(needs_layout_passes=False),
    )(seeds_padded)


def _blend_body(emb_ref, m_ref, tok_ref, out_ref):
    i = pl.program_id(0)
    m = m_ref[pl.ds(i * ROWS_BLK, ROWS_BLK)].reshape(ROWS_BLK, 1)
    out_ref[...] = jnp.where(m == 0.0, tok_ref[...], emb_ref[...])


def kernel(embeds, seeds, mask_token):
    mask = _build_mask(seeds)

    out = pl.pallas_call(
        _blend_body,
        grid=(NBLK,),
        in_specs=[
            pl.BlockSpec((ROWS_BLK, D), lambda i: (i, 0)),
            pl.BlockSpec((NPAD,), lambda i: (0,)),
            pl.BlockSpec((1, D), lambda i: (0, 0)),
        ],
        out_specs=pl.BlockSpec((ROWS_BLK, D), lambda i: (i, 0)),
        out_shape=jax.ShapeDtypeStruct((N, D), jnp.float32),
    )(embeds, mask, mask_token)
    return (out, seeds)


# speedup vs baseline: 1.0245x; 1.0245x over previous
import functools

import jax
import jax.numpy as jnp
from jax import lax
from jax.experimental import pallas as pl
from jax.experimental.pallas import tpu as pltpu
from jax.experimental.pallas import tpu_sc as plsc

N = 100000
D = 512
S = 15000

L = 16
NC = 2
NS = 16
NW = NC * NS
CHUNK = 3200
NPAD = NW * CHUNK
CHUNK2 = NPAD // NS
S_HALF = 7504
S_OFF = S - S_HALF

ROWS_BLK = 4096
NBLK = (N + ROWS_BLK - 1) // ROWS_BLK


def _mask_sc_body(seeds_hbm, mask_a_hbm, mask_b_hbm, seeds_v, mask_v):
    c = lax.axis_index("c")
    base = lax.axis_index("s") * CHUNK2
    off = pl.multiple_of(c * S_OFF, 8)
    pltpu.sync_copy(seeds_hbm.at[pl.ds(off, S_HALF)], seeds_v)

    ones = jnp.ones((L,), jnp.float32)
    zeros = jnp.zeros((L,), jnp.float32)

    @plsc.parallel_loop(0, CHUNK2, L, unroll=8)
    def _init(i):
        mask_v[pl.ds(i, L)] = ones

    @plsc.parallel_loop(0, S_HALF, L, unroll=8)
    def _scan(g):
        s = seeds_v[pl.ds(g, L)]
        local = s - base
        inr = (local >= 0) & (local < CHUNK2)
        idx = jnp.where(inr, local, 0)
        plsc.store_scatter(mask_v, [idx], zeros, mask=inr)

    @pl.when(c == 0)
    def _store_a():
        pltpu.sync_copy(mask_v, mask_a_hbm.at[pl.ds(base, CHUNK2)])

    @pl.when(c == 1)
    def _store_b():
        pltpu.sync_copy(mask_v, mask_b_hbm.at[pl.ds(base, CHUNK2)])


def _build_mask(seeds_padded):
    mesh = plsc.VectorSubcoreMesh(core_axis_name="c", subcore_axis_name="s")
    return pl.kernel(
        _mask_sc_body,
        mesh=mesh,
        out_type=(jax.ShapeDtypeStruct((NPAD,), jnp.float32),
                  jax.ShapeDtypeStruct((NPAD,), jnp.float32)),
        scratch_types=[
            pltpu.VMEM((S_HALF,), jnp.int32),
            pltpu.VMEM((CHUNK2,), jnp.float32),
        ],
        compiler_params=pltpu.CompilerParams(needs_layout_passes=False),
    )(seeds_padded)


def _blend_body(emb_ref, ma_ref, mb_ref, tok_ref, out_ref):
    i = pl.program_id(0)
    sl = pl.ds(i * ROWS_BLK, ROWS_BLK)
    m = (ma_ref[sl] * mb_ref[sl]).reshape(ROWS_BLK, 1)
    out_ref[...] = jnp.where(m == 0.0, tok_ref[...], emb_ref[...])


def kernel(embeds, seeds, mask_token):
    mask_a, mask_b = _build_mask(seeds)

    out = pl.pallas_call(
        _blend_body,
        grid=(NBLK,),
        in_specs=[
            pl.BlockSpec((ROWS_BLK, D), lambda i: (i, 0)),
            pl.BlockSpec((NPAD,), lambda i: (0,)),
            pl.BlockSpec((NPAD,), lambda i: (0,)),
            pl.BlockSpec((1, D), lambda i: (0, 0)),
        ],
        out_specs=pl.BlockSpec((ROWS_BLK, D), lambda i: (i, 0)),
        out_shape=jax.ShapeDtypeStruct((N, D), jnp.float32),
    )(embeds, mask_a, mask_b, mask_token)
    return (out, seeds)
